# baseline (device time: 11961 ns/iter reference)
import jax
import jax.numpy as jnp
from jax import lax
from jax.experimental import pallas as pl
from jax.experimental.pallas import tpu as pltpu


def kernel(x, W, labels):
    T, D = x.shape
    _, V = W.shape
    labels2d = labels.reshape(1, T)

    def body(x_ref, w_ref, lab_ref, out_ref, send_buf, recv_buf, send_sem, recv_sem):
        my_x = lax.axis_index("x")
        my_y = lax.axis_index("y")
        my_z = lax.axis_index("z")
        peer = (my_x, 1 - my_y, my_z)

        barrier = pltpu.get_barrier_semaphore()
        pl.semaphore_signal(
            barrier, inc=1, device_id=peer, device_id_type=pl.DeviceIdType.MESH
        )
        pl.semaphore_wait(barrier, 1)

        xb = x_ref[:, :].astype(jnp.bfloat16)
        wb = w_ref[:, :].astype(jnp.bfloat16)
        logits_t = lax.dot_general(
            wb, xb,
            dimension_numbers=(((0,), (1,)), ((), ())),
            preferred_element_type=jnp.float32,
        )

        m = jnp.max(logits_t, axis=0)
        s = jnp.sum(jnp.exp(logits_t - m[None, :]), axis=0)
        local_label = lab_ref[0, :] - my_y * V
        row_ids = lax.broadcasted_iota(jnp.int32, (V, T), 0)
        hit = row_ids == local_label[None, :]
        c = jnp.sum(jnp.where(hit, logits_t, 0.0), axis=0)

        send_buf[0, :] = m
        send_buf[1, :] = s
        send_buf[2, :] = c

        rdma = pltpu.make_async_remote_copy(
            src_ref=send_buf,
            dst_ref=recv_buf,
            send_sem=send_sem,
            recv_sem=recv_sem,
            device_id=peer,
            device_id_type=pl.DeviceIdType.MESH,
        )
        rdma.start()
        rdma.wait()

        pm = recv_buf[0, :]
        ps = recv_buf[1, :]
        pc = recv_buf[2, :]
        mm = jnp.maximum(m, pm)
        tot = s * jnp.exp(m - mm) + ps * jnp.exp(pm - mm)
        out_ref[0, :] = mm + jnp.log(tot) - (c + pc)

    out = pl.pallas_call(
        body,
        out_shape=jax.ShapeDtypeStruct((1, T), jnp.float32),
        in_specs=[
            pl.BlockSpec(memory_space=pltpu.VMEM),
            pl.BlockSpec(memory_space=pltpu.VMEM),
            pl.BlockSpec(memory_space=pltpu.VMEM),
        ],
        out_specs=pl.BlockSpec(memory_space=pltpu.VMEM),
        scratch_shapes=[
            pltpu.VMEM((8, T), jnp.float32),
            pltpu.VMEM((8, T), jnp.float32),
            pltpu.SemaphoreType.DMA,
            pltpu.SemaphoreType.DMA,
        ],
        compiler_params=pltpu.CompilerParams(collective_id=0),
    )(x, W, labels2d)
    return out.reshape(T)
